# baseline (speedup 1.0000x reference)
import functools

import jax
import jax.numpy as jnp
from jax import lax
from jax.experimental import pallas as pl
from jax.experimental.pallas import tpu as pltpu
from jax.experimental.pallas import tpu_sc as plsc

_LANES = 16
_CHUNK = 128


@functools.lru_cache(maxsize=None)
def _build(batch, vocab, dim):
    info = plsc.get_sparse_core_info()
    nc, ns = info.num_cores, info.num_subcores
    nw = nc * ns
    b_per_w = batch // nw
    nch = b_per_w // _CHUNK
    ngrp = _CHUNK // _LANES

    mesh = plsc.VectorSubcoreMesh(core_axis_name="c", subcore_axis_name="s")

    @functools.partial(
        pl.kernel,
        mesh=mesh,
        out_type=jax.ShapeDtypeStruct((batch,), jnp.float32),
        compiler_params=pltpu.CompilerParams(
            needs_layout_passes=False, use_tc_tiling_on_sc=False),
        scratch_types=[
            pltpu.VMEM((nch, _CHUNK), jnp.int32),
            pltpu.VMEM((nch, _CHUNK), jnp.int32),
            pltpu.VMEM((2, dim, _CHUNK), jnp.int32),
            pltpu.VMEM((2, dim, _CHUNK), jnp.int32),
            pltpu.VMEM((2, dim, _CHUNK), jnp.float32),
            pltpu.VMEM((2, dim, _CHUNK), jnp.float32),
            pltpu.VMEM((b_per_w,), jnp.float32),
            pltpu.SemaphoreType.DMA,
            pltpu.SemaphoreType.DMA,
        ],
    )
    def word2vec_sc(af_hbm, bf_hbm, ci_hbm, xi_hbm, out_hbm,
                    ci_v, xi_v, ia_v, ib_v, a_v, b_v, o_v, sem0, sem1):
        wid = lax.axis_index("s") * nc + lax.axis_index("c")
        base = wid * b_per_w
        pltpu.sync_copy(ci_hbm.at[wid], ci_v)
        pltpu.sync_copy(xi_hbm.at[wid], xi_v)
        sems = (sem0, sem1)

        def fire(j):
            buf = j % 2
            sem = sems[buf]
            for iv, cv in ((ia_v, ci_v), (ib_v, xi_v)):
                def gen(d, carry, _iv=iv, _cv=cv, _buf=buf):
                    step = d * vocab
                    for g in range(ngrp):
                        sl = pl.ds(g * _LANES, _LANES)
                        _iv[_buf, d, sl] = cv[j, sl] + step
                    return carry

                lax.fori_loop(0, dim, gen, 0)
            for hv, iv, dv in ((af_hbm, ia_v, a_v), (bf_hbm, ib_v, b_v)):
                def launch(d, carry, _hv=hv, _iv=iv, _dv=dv, _buf=buf,
                           _sem=sem):
                    pltpu.async_copy(
                        _hv.at[_iv.at[_buf, d]], _dv.at[_buf, d], _sem)
                    return carry

                lax.fori_loop(0, dim, launch, 0)

        def drain(j):
            buf = j % 2
            sem = sems[buf]
            for hv, iv, dv in ((af_hbm, ia_v, a_v), (bf_hbm, ib_v, b_v)):
                def dwait(d, carry, _hv=hv, _iv=iv, _dv=dv, _buf=buf,
                          _sem=sem):
                    pltpu.make_async_copy(
                        _hv.at[_iv.at[_buf, d]], _dv.at[_buf, d], _sem).wait()
                    return carry

                lax.fori_loop(0, dim, dwait, 0)

        fire(0)
        for j in range(nch):
            if j + 1 < nch:
                fire(j + 1)
            drain(j)
            buf = j % 2

            def body(d, accs, _buf=buf):
                new = []
                for g in range(ngrp):
                    sl = pl.ds(g * _LANES, _LANES)
                    new.append(accs[g] + a_v[_buf, d, sl] * b_v[_buf, d, sl])
                return tuple(new)

            zero = jnp.zeros((_LANES,), dtype=jnp.float32)
            accs = lax.fori_loop(0, dim, body, (zero,) * ngrp)
            for g in range(ngrp):
                o_v[pl.ds(j * _CHUNK + g * _LANES, _LANES)] = accs[g]

        pltpu.sync_copy(o_v, out_hbm.at[pl.ds(base, b_per_w)])

    return word2vec_sc


def kernel(center, context, embeddings, output_embeddings):
    batch = center.shape[0]
    vocab, dim = embeddings.shape
    info = plsc.get_sparse_core_info()
    nw = info.num_cores * info.num_subcores
    b_per_w = batch // nw
    nch = b_per_w // _CHUNK
    ci = center.astype(jnp.int32).reshape(nw, nch, _CHUNK)
    xi = context.astype(jnp.int32).reshape(nw, nch, _CHUNK)
    af = jnp.swapaxes(embeddings, 0, 1).reshape(-1)
    bf = jnp.swapaxes(output_embeddings, 0, 1).reshape(-1)
    fn = _build(batch, vocab, dim)
    return fn(af, bf, ci, xi)


# speedup vs baseline: 1.0018x; 1.0018x over previous
import functools

import jax
import jax.numpy as jnp
from jax import lax
from jax.experimental import pallas as pl
from jax.experimental.pallas import tpu as pltpu
from jax.experimental.pallas import tpu_sc as plsc

_LANES = 16
_CHUNK = 128


@functools.lru_cache(maxsize=None)
def _build(batch, vocab, dim):
    info = plsc.get_sparse_core_info()
    nc, ns = info.num_cores, info.num_subcores
    nw = nc * ns
    b_per_w = batch // nw
    nch = b_per_w // _CHUNK
    ngrp = _CHUNK // _LANES
    nflat = dim * _CHUNK

    mesh = plsc.VectorSubcoreMesh(core_axis_name="c", subcore_axis_name="s")

    @functools.partial(
        pl.kernel,
        mesh=mesh,
        out_type=jax.ShapeDtypeStruct((batch,), jnp.float32),
        compiler_params=pltpu.CompilerParams(
            needs_layout_passes=False, use_tc_tiling_on_sc=False),
        scratch_types=[
            pltpu.VMEM((nch, _CHUNK), jnp.int32),
            pltpu.VMEM((nch, _CHUNK), jnp.int32),
            pltpu.VMEM((2, nflat), jnp.int32),
            pltpu.VMEM((2, nflat), jnp.int32),
            pltpu.VMEM((2, nflat), jnp.float32),
            pltpu.VMEM((2, nflat), jnp.float32),
            pltpu.VMEM((b_per_w,), jnp.float32),
            pltpu.SemaphoreType.DMA,
            pltpu.SemaphoreType.DMA,
        ],
    )
    def word2vec_sc(af_hbm, bf_hbm, ci_hbm, xi_hbm, out_hbm,
                    ci_v, xi_v, ia_v, ib_v, a_v, b_v, o_v, sem0, sem1):
        wid = lax.axis_index("s") * nc + lax.axis_index("c")
        base = wid * b_per_w
        pltpu.sync_copy(ci_hbm.at[wid], ci_v)
        pltpu.sync_copy(xi_hbm.at[wid], xi_v)
        sems = (sem0, sem1)

        def fire(j):
            buf = j % 2
            sem = sems[buf]
            for iv, cv in ((ia_v, ci_v), (ib_v, xi_v)):
                def gen(d, carry, _iv=iv, _buf=buf):
                    for g in range(ngrp):
                        sl = pl.ds(d * _CHUNK + g * _LANES, _LANES)
                        _iv[_buf, sl] = carry[g]
                    return tuple(c + vocab for c in carry)

                c0 = tuple(
                    cv[j, pl.ds(g * _LANES, _LANES)] for g in range(ngrp))
                lax.fori_loop(0, dim, gen, c0)
            pltpu.async_copy(af_hbm.at[ia_v.at[buf]], a_v.at[buf], sem)
            pltpu.async_copy(bf_hbm.at[ib_v.at[buf]], b_v.at[buf], sem)

        def drain(j):
            buf = j % 2
            sem = sems[buf]
            pltpu.make_async_copy(
                af_hbm.at[ia_v.at[buf]], a_v.at[buf], sem).wait()
            pltpu.make_async_copy(
                bf_hbm.at[ib_v.at[buf]], b_v.at[buf], sem).wait()

        fire(0)
        for j in range(nch):
            if j + 1 < nch:
                fire(j + 1)
            drain(j)
            buf = j % 2

            def body(d, accs, _buf=buf):
                new = []
                for g in range(ngrp):
                    sl = pl.ds(d * _CHUNK + g * _LANES, _LANES)
                    new.append(accs[g] + a_v[_buf, sl] * b_v[_buf, sl])
                return tuple(new)

            zero = jnp.zeros((_LANES,), dtype=jnp.float32)
            accs = lax.fori_loop(0, dim, body, (zero,) * ngrp)
            for g in range(ngrp):
                o_v[pl.ds(j * _CHUNK + g * _LANES, _LANES)] = accs[g]

        pltpu.sync_copy(o_v, out_hbm.at[pl.ds(base, b_per_w)])

    return word2vec_sc


def kernel(center, context, embeddings, output_embeddings):
    batch = center.shape[0]
    vocab, dim = embeddings.shape
    info = plsc.get_sparse_core_info()
    nw = info.num_cores * info.num_subcores
    b_per_w = batch // nw
    nch = b_per_w // _CHUNK
    ci = center.astype(jnp.int32).reshape(nw, nch, _CHUNK)
    xi = context.astype(jnp.int32).reshape(nw, nch, _CHUNK)
    af = jnp.swapaxes(embeddings, 0, 1).reshape(-1)
    bf = jnp.swapaxes(output_embeddings, 0, 1).reshape(-1)
    fn = _build(batch, vocab, dim)
    return fn(af, bf, ci, xi)


# speedup vs baseline: 1.0021x; 1.0004x over previous
import functools

import jax
import jax.numpy as jnp
from jax import lax
from jax.experimental import pallas as pl
from jax.experimental.pallas import tpu as pltpu
from jax.experimental.pallas import tpu_sc as plsc

_LANES = 16
_CHUNK = 16


@functools.lru_cache(maxsize=None)
def _build(batch, vocab, dim):
    info = plsc.get_sparse_core_info()
    nc, ns = info.num_cores, info.num_subcores
    nw = nc * ns
    b_per_w = batch // nw
    nch = b_per_w // _CHUNK
    vblk = vocab // _LANES
    nrow = dim * _CHUNK

    mesh = plsc.VectorSubcoreMesh(core_axis_name="c", subcore_axis_name="s")

    @functools.partial(
        pl.kernel,
        mesh=mesh,
        out_type=jax.ShapeDtypeStruct((batch,), jnp.float32),
        compiler_params=pltpu.CompilerParams(
            needs_layout_passes=False, use_tc_tiling_on_sc=False),
        scratch_types=[
            pltpu.VMEM((b_per_w,), jnp.int32),
            pltpu.VMEM((b_per_w,), jnp.int32),
            pltpu.VMEM((2, nrow), jnp.int32),
            pltpu.VMEM((2, nrow), jnp.int32),
            pltpu.VMEM((2, nrow, _LANES), jnp.float32),
            pltpu.VMEM((2, nrow, _LANES), jnp.float32),
            pltpu.VMEM((b_per_w,), jnp.float32),
            pltpu.SemaphoreType.DMA,
            pltpu.SemaphoreType.DMA,
        ],
    )
    def word2vec_sc(ag_hbm, bg_hbm, ci_hbm, xi_hbm, out_hbm,
                    ci_v, xi_v, ia_v, ib_v, a_v, b_v, o_v, sem0, sem1):
        wid = lax.axis_index("s") * nc + lax.axis_index("c")
        base = wid * b_per_w
        pltpu.sync_copy(ci_hbm.at[wid], ci_v)
        pltpu.sync_copy(xi_hbm.at[wid], xi_v)
        sems = (sem0, sem1)

        def fire(j):
            buf = j % 2
            sem = sems[buf]
            for iv, cv in ((ia_v, ci_v), (ib_v, xi_v)):
                c = cv[pl.ds(j * _CHUNK, _CHUNK)]
                base_row = lax.shift_right_logical(c, 4)

                def gen(d, carry, _iv=iv, _buf=buf):
                    _iv[_buf, pl.ds(d * _CHUNK, _CHUNK)] = carry
                    return carry + vblk

                lax.fori_loop(0, dim, gen, base_row)
            pltpu.async_copy(ag_hbm.at[ia_v.at[buf]], a_v.at[buf], sem)
            pltpu.async_copy(bg_hbm.at[ib_v.at[buf]], b_v.at[buf], sem)

        def drain(j):
            buf = j % 2
            sem = sems[buf]
            pltpu.make_async_copy(
                ag_hbm.at[ia_v.at[buf]], a_v.at[buf], sem).wait()
            pltpu.make_async_copy(
                bg_hbm.at[ib_v.at[buf]], b_v.at[buf], sem).wait()

        lane_iota = jnp.arange(_LANES, dtype=jnp.int32)
        fifteen = jnp.full((_LANES,), 15, dtype=jnp.int32)

        fire(0)
        for j in range(nch):
            if j + 1 < nch:
                fire(j + 1)
            drain(j)
            buf = j % 2
            la = jnp.bitwise_and(ci_v[pl.ds(j * _CHUNK, _CHUNK)], fifteen)
            lb = jnp.bitwise_and(xi_v[pl.ds(j * _CHUNK, _CHUNK)], fifteen)

            def body(d, acc, _buf=buf):
                rows = d * _CHUNK + lane_iota
                va = plsc.load_gather(a_v.at[_buf], [rows, la])
                vb = plsc.load_gather(b_v.at[_buf], [rows, lb])
                return acc + va * vb

            zero = jnp.zeros((_LANES,), dtype=jnp.float32)
            acc = lax.fori_loop(0, dim, body, zero)
            o_v[pl.ds(j * _CHUNK, _CHUNK)] = acc

        pltpu.sync_copy(o_v, out_hbm.at[pl.ds(base, b_per_w)])

    return word2vec_sc


def kernel(center, context, embeddings, output_embeddings):
    batch = center.shape[0]
    vocab, dim = embeddings.shape
    info = plsc.get_sparse_core_info()
    nw = info.num_cores * info.num_subcores
    b_per_w = batch // nw
    ci = center.astype(jnp.int32).reshape(nw, b_per_w)
    xi = context.astype(jnp.int32).reshape(nw, b_per_w)
    ag = jnp.swapaxes(embeddings, 0, 1).reshape(-1, _LANES)
    bg = jnp.swapaxes(output_embeddings, 0, 1).reshape(-1, _LANES)
    fn = _build(batch, vocab, dim)
    return fn(ag, bg, ci, xi)


# speedup vs baseline: 9.1394x; 9.1198x over previous
import functools

import jax
import jax.numpy as jnp
from jax import lax
from jax.experimental import pallas as pl
from jax.experimental.pallas import tpu as pltpu
from jax.experimental.pallas import tpu_sc as plsc

_LANES = 16
_CHUNK = 128


@functools.lru_cache(maxsize=None)
def _build(batch, vocab, dim):
    info = plsc.get_sparse_core_info()
    nc, ns = info.num_cores, info.num_subcores
    nw = nc * ns
    b_per_w = batch // nw
    nch = b_per_w // _CHUNK
    nslice = dim // _LANES

    mesh = plsc.VectorSubcoreMesh(core_axis_name="c", subcore_axis_name="s")

    @functools.partial(
        pl.kernel,
        mesh=mesh,
        out_type=jax.ShapeDtypeStruct((batch,), jnp.float32),
        compiler_params=pltpu.CompilerParams(
            needs_layout_passes=False, use_tc_tiling_on_sc=False),
        scratch_types=[
            pltpu.VMEM((nch, _CHUNK), jnp.int32),
            pltpu.VMEM((nch, _CHUNK), jnp.int32),
            pltpu.VMEM((2, _CHUNK, dim), jnp.float32),
            pltpu.VMEM((2, _CHUNK, dim), jnp.float32),
            pltpu.VMEM((b_per_w,), jnp.float32),
            pltpu.SemaphoreType.DMA,
            pltpu.SemaphoreType.DMA,
            pltpu.SemaphoreType.DMA,
            pltpu.SemaphoreType.DMA,
        ],
    )
    def word2vec_sc(emb_hbm, oemb_hbm, ci_hbm, xi_hbm, out_hbm,
                    ci_v, xi_v, a_v, b_v, o_v, sa0, sa1, sb0, sb1):
        wid = lax.axis_index("s") * nc + lax.axis_index("c")
        base = wid * b_per_w
        pltpu.sync_copy(ci_hbm.at[wid], ci_v)
        pltpu.sync_copy(xi_hbm.at[wid], xi_v)
        sas = (sa0, sa1)
        sbs = (sb0, sb1)

        def start(j):
            buf = j % 2
            ha = pltpu.async_copy(emb_hbm.at[ci_v.at[j]], a_v.at[buf], sas[buf])
            hb = pltpu.async_copy(oemb_hbm.at[xi_v.at[j]], b_v.at[buf], sbs[buf])
            return ha, hb

        handles = [None] * nch
        handles[0] = start(0)
        for j in range(nch):
            if j + 1 < nch:
                handles[j + 1] = start(j + 1)
            ha, hb = handles[j]
            ha.wait()
            hb.wait()
            buf = j % 2

            lane_iota = jnp.arange(_LANES, dtype=jnp.int32)

            def grp(g, carry, _buf=buf, _j=j):
                out = jnp.zeros((_LANES,), dtype=jnp.float32)
                for rr in range(_LANES):
                    r = g * _LANES + rr
                    acc = a_v[_buf, r, 0:_LANES] * b_v[_buf, r, 0:_LANES]
                    for c in range(1, nslice):
                        lo = c * _LANES
                        acc = acc + (a_v[_buf, r, lo:lo + _LANES]
                                     * b_v[_buf, r, lo:lo + _LANES])
                    s = jnp.sum(acc)
                    out = jnp.where(lane_iota == rr, s, out)
                o_v[pl.ds(_j * _CHUNK + g * _LANES, _LANES)] = out
                return carry

            lax.fori_loop(0, _CHUNK // _LANES, grp, 0)

        pltpu.sync_copy(o_v, out_hbm.at[pl.ds(base, b_per_w)])

    return word2vec_sc


def kernel(center, context, embeddings, output_embeddings):
    batch = center.shape[0]
    vocab, dim = embeddings.shape
    info = plsc.get_sparse_core_info()
    nw = info.num_cores * info.num_subcores
    b_per_w = batch // nw
    nch = b_per_w // _CHUNK
    ci = center.astype(jnp.int32).reshape(nw, nch, _CHUNK)
    xi = context.astype(jnp.int32).reshape(nw, nch, _CHUNK)
    fn = _build(batch, vocab, dim)
    return fn(embeddings, output_embeddings, ci, xi)
